# Initial kernel scaffold; baseline (speedup 1.0000x reference)
import jax
import jax.numpy as jnp
from jax.experimental import pallas as pl


def kernel(x, edge_index, edge_attr, batch, W_edge_init, b_edge_init, W_conv0, b_conv0, W_conv1, b_conv1, W_conv2, b_conv2, W_e2n, b_e2n, W_ffn, b_ffn):
    raise NotImplementedError("write your pallas kernel here")


# speedup vs baseline: 2.8333x; 2.8333x over previous
import functools

import jax
import jax.numpy as jnp
from jax import lax
from jax.experimental import pallas as pl
from jax.experimental.pallas import tpu as pltpu
from jax.experimental.pallas import tpu_sc as plsc

_N = 10000
_NP = 10240
_E = 640000
_H = 128
_G = 512

_NC = 2
_NS = 16
_NW = _NC * _NS

_CS = 200
_CG = 200
_CX = 400

_BE = 1024
_BN = 1000

_sc_mesh = plsc.VectorSubcoreMesh(core_axis_name="c", subcore_axis_name="s")


def _gather_x_body(x_hbm, row_hbm, out_hbm, idx_v, rows_v, sem):
    wid = lax.axis_index("s") * _NC + lax.axis_index("c")
    per_w = _E // _NW
    base = wid * per_w

    def step(k, _):
        off = base + k * _CX
        pltpu.sync_copy(row_hbm.at[pl.ds(off, _CX)], idx_v)
        pltpu.async_copy(x_hbm.at[idx_v], rows_v, sem).wait()
        pltpu.sync_copy(rows_v, out_hbm.at[pl.ds(off, _CX), :])
        return _

    lax.fori_loop(0, per_w // _CX, step, 0)


_gather_x = functools.partial(
    pl.kernel,
    out_type=jax.ShapeDtypeStruct((_E, _H), jnp.float32),
    mesh=_sc_mesh,
    scratch_types=[
        pltpu.VMEM((_CX,), jnp.int32),
        pltpu.VMEM((_CX, _H), jnp.float32),
        pltpu.SemaphoreType.DMA,
    ],
)(_gather_x_body)


def _sc_layer_body(h_hbm, col_hbm, row_hbm, zeros_hbm, m_hbm,
                   table_sh, idx_v, rows_v, sem):
    cid = lax.axis_index("c")
    sid = lax.axis_index("s")

    rows_per_tile = _NP // _NS
    pltpu.sync_copy(zeros_hbm.at[pl.ds(sid * rows_per_tile, rows_per_tile), :],
                    table_sh.at[pl.ds(sid * rows_per_tile, rows_per_tile), :])
    plsc.subcore_barrier()

    per_tile = _E // _NS
    sbase = sid * per_tile

    def sstep(k, _):
        off = sbase + k * _CS
        pltpu.sync_copy(col_hbm.at[pl.ds(off, _CS)], idx_v)
        pltpu.sync_copy(h_hbm.at[pl.ds(off, _CS), :], rows_v)
        pltpu.sync_copy(rows_v, table_sh.at[idx_v], add=True)
        return _

    lax.fori_loop(0, per_tile // _CS, sstep, 0)
    plsc.subcore_barrier()

    wid = sid * _NC + cid
    per_w = _E // _NW
    gbase = wid * per_w

    def gstep(k, _):
        off = gbase + k * _CG
        pltpu.sync_copy(row_hbm.at[pl.ds(off, _CG)], idx_v)
        pltpu.async_copy(table_sh.at[idx_v], rows_v, sem).wait()
        pltpu.sync_copy(rows_v, m_hbm.at[pl.ds(off, _CG), :])
        return _

    lax.fori_loop(0, per_w // _CG, gstep, 0)


_sc_layer = functools.partial(
    pl.kernel,
    out_type=jax.ShapeDtypeStruct((_E, _H), jnp.float32),
    mesh=_sc_mesh,
    scratch_types=[
        pltpu.VMEM_SHARED((_NP, _H), jnp.float32),
        pltpu.VMEM((_CS,), jnp.int32),
        pltpu.VMEM((_CS, _H), jnp.float32),
        pltpu.SemaphoreType.DMA,
    ],
)(_sc_layer_body)


def _sc_scatter_body(h_hbm, col_hbm, zeros_hbm, out_hbm, table_sh, idx_v, rows_v):
    cid = lax.axis_index("c")
    sid = lax.axis_index("s")

    rows_per_tile = _NP // _NS
    pltpu.sync_copy(zeros_hbm.at[pl.ds(sid * rows_per_tile, rows_per_tile), :],
                    table_sh.at[pl.ds(sid * rows_per_tile, rows_per_tile), :])
    plsc.subcore_barrier()

    wid = sid * _NC + cid
    per_w = _E // _NW
    base = wid * per_w

    def sstep(k, _):
        off = base + k * _CS
        pltpu.sync_copy(col_hbm.at[pl.ds(off, _CS)], idx_v)
        pltpu.sync_copy(h_hbm.at[pl.ds(off, _CS), :], rows_v)
        pltpu.sync_copy(rows_v, table_sh.at[idx_v], add=True)
        return _

    lax.fori_loop(0, per_w // _CS, sstep, 0)
    plsc.subcore_barrier()

    pltpu.sync_copy(table_sh.at[pl.ds(sid * rows_per_tile, rows_per_tile), :],
                    out_hbm.at[cid, pl.ds(sid * rows_per_tile, rows_per_tile), :])


_sc_scatter = functools.partial(
    pl.kernel,
    out_type=jax.ShapeDtypeStruct((_NC, _NP, _H), jnp.float32),
    mesh=_sc_mesh,
    scratch_types=[
        pltpu.VMEM_SHARED((_NP, _H), jnp.float32),
        pltpu.VMEM((_CS,), jnp.int32),
        pltpu.VMEM((_CS, _H), jnp.float32),
    ],
)(_sc_scatter_body)


def _edge_init_body(xg_ref, ea_ref, w_ref, b_ref, out_ref):
    q = jnp.concatenate([xg_ref[...], ea_ref[...]], axis=1)
    acc = jnp.dot(q, w_ref[...], preferred_element_type=jnp.float32)
    out_ref[...] = jnp.maximum(acc + b_ref[...], 0.0)


def _conv_body(m_ref, h_ref, h0_ref, w_ref, b_ref, out_ref):
    h = h_ref[...]
    up = pltpu.roll(h, _BE - 1, 0)
    dn = pltpu.roll(h, 1, 0)
    par = lax.broadcasted_iota(jnp.int32, h.shape, 0) % 2
    rev = jnp.where(par == 0, up, dn)
    t = m_ref[...] - rev
    acc = jnp.dot(t, w_ref[...], preferred_element_type=jnp.float32)
    out_ref[...] = jnp.maximum(acc + b_ref[...] + h0_ref[...], 0.0)


def _final_body(x_ref, p0_ref, p1_ref, bat_ref, w_ref, wf_ref,
                be_ref, bff_ref, out_ref, acc_ref):
    s = p0_ref[...] + p1_ref[...]
    q = jnp.concatenate([x_ref[...], s], axis=1)
    acc = jnp.dot(q, w_ref[...], preferred_element_type=jnp.float32)
    hn = jnp.maximum(acc + be_ref[...], 0.0)
    gi = lax.broadcasted_iota(jnp.int32, (_BN, _G), 1)
    onehot = (bat_ref[...] == gi).astype(jnp.float32)
    pooled = lax.dot_general(onehot, hn, (((0,), (0,)), ((), ())),
                             preferred_element_type=jnp.float32)

    @pl.when(pl.program_id(0) == 0)
    def _():
        acc_ref[...] = jnp.zeros((_G, _H), jnp.float32)

    acc_ref[...] += pooled

    @pl.when(pl.program_id(0) == _N // _BN - 1)
    def _():
        out_ref[...] = (jnp.dot(acc_ref[...], wf_ref[...],
                                preferred_element_type=jnp.float32)
                        + bff_ref[...])


def _edge_block(i):
    return (i, 0)


def _fixed(i):
    return (0, 0)


_eb_spec = pl.BlockSpec((_BE, _H), _edge_block)


def _tc_edge_init(xg, ea, w, b):
    return pl.pallas_call(
        _edge_init_body,
        grid=(_E // _BE,),
        in_specs=[
            _eb_spec,
            pl.BlockSpec((_BE, 16), _edge_block),
            pl.BlockSpec((_H + 16, _H), _fixed),
            pl.BlockSpec((1, _H), _fixed),
        ],
        out_specs=_eb_spec,
        out_shape=jax.ShapeDtypeStruct((_E, _H), jnp.float32),
    )(xg, ea, w, b)


def _tc_conv(m, h, h0, w, b):
    return pl.pallas_call(
        _conv_body,
        grid=(_E // _BE,),
        in_specs=[
            _eb_spec, _eb_spec, _eb_spec,
            pl.BlockSpec((_H, _H), _fixed),
            pl.BlockSpec((1, _H), _fixed),
        ],
        out_specs=_eb_spec,
        out_shape=jax.ShapeDtypeStruct((_E, _H), jnp.float32),
    )(m, h, h0, w, b)


def _tc_final(x, p0, p1, bat2d, w, wf, be, bff):
    nb_spec = pl.BlockSpec((_BN, _H), _edge_block)
    out = pl.pallas_call(
        _final_body,
        grid=(_N // _BN,),
        in_specs=[
            nb_spec, nb_spec, nb_spec,
            pl.BlockSpec((_BN, 1), _edge_block),
            pl.BlockSpec((2 * _H, _H), _fixed),
            pl.BlockSpec((_H, 1), _fixed),
            pl.BlockSpec((1, _H), _fixed),
            pl.BlockSpec((1, 1), _fixed),
        ],
        out_specs=pl.BlockSpec((_G, 1), _fixed),
        out_shape=jax.ShapeDtypeStruct((_G, 1), jnp.float32),
        scratch_shapes=[pltpu.VMEM((_G, _H), jnp.float32)],
    )(x, p0, p1, bat2d, w, wf, be, bff)
    return out.reshape(_G)


def kernel(x, edge_index, edge_attr, batch,
           W_edge_init, b_edge_init,
           W_conv0, b_conv0, W_conv1, b_conv1, W_conv2, b_conv2,
           W_e2n, b_e2n, W_ffn, b_ffn):
    row = edge_index[0]
    col = edge_index[1]
    zeros_n = jnp.zeros((_NP, _H), jnp.float32)

    be0 = b_edge_init.reshape(1, _H)

    xg = _gather_x(x, row)
    h0 = _tc_edge_init(xg, edge_attr, W_edge_init.T, be0)

    h = h0
    for w, b in ((W_conv0, b_conv0), (W_conv1, b_conv1), (W_conv2, b_conv2)):
        m = _sc_layer(h, col, row, zeros_n)
        h = _tc_conv(m, h, h0, w.T, b.reshape(1, _H))

    parts = _sc_scatter(h, col, zeros_n)

    out = _tc_final(x, parts[0, :_N], parts[1, :_N], batch.reshape(_N, 1),
                    W_e2n.T, W_ffn.reshape(_H, 1),
                    b_e2n.reshape(1, _H), b_ffn.reshape(1, 1))
    return out


# speedup vs baseline: 3.7162x; 1.3116x over previous
import functools

import jax
import jax.numpy as jnp
from jax import lax
from jax.experimental import pallas as pl
from jax.experimental.pallas import tpu as pltpu
from jax.experimental.pallas import tpu_sc as plsc

_N = 10000
_NP = 10240
_E = 640000
_H = 128
_G = 512

_NC = 2
_NS = 16
_NW = _NC * _NS

_CS = 160
_CG = 160
_CX = 400

_BE = 1024
_BN = 1000

_sc_mesh = plsc.VectorSubcoreMesh(core_axis_name="c", subcore_axis_name="s")


def _gather_pipe(table, row_hbm, out_hbm, base, nch, csz,
                 idx, rows, isem, gsem, ssem):
    def istart(c, b):
        pltpu.async_copy(row_hbm.at[pl.ds(base + c * csz, csz)], idx[b], isem[b])

    def iwait(b):
        pltpu.make_async_copy(row_hbm.at[pl.ds(base, csz)], idx[b], isem[b]).wait()

    def swait(b):
        pltpu.make_async_copy(rows[b], out_hbm.at[pl.ds(base, csz), :], ssem[b]).wait()

    def halfstep(c, b, k2):
        @pl.when(c < nch)
        def _():
            iwait(b)
            @pl.when(k2 >= 1)
            def _():
                swait(b)
            pltpu.async_copy(table.at[idx[b]], rows[b], gsem).wait()
            @pl.when(c + 2 < nch)
            def _():
                istart(c + 2, b)
            pltpu.async_copy(rows[b], out_hbm.at[pl.ds(base + c * csz, csz), :],
                             ssem[b])

    istart(0, 0)
    @pl.when(nch > 1)
    def _():
        istart(1, 1)

    def body(k2, _):
        halfstep(2 * k2, 0, k2)
        halfstep(2 * k2 + 1, 1, k2)
        return _

    lax.fori_loop(0, (nch + 1) // 2, body, 0)
    swait(0)
    @pl.when(nch > 1)
    def _():
        swait(1)


def _scatter_pipe(h_hbm, col_hbm, table_sh, base, nch, csz,
                  idx, rows, isem):
    def start(c, b):
        off = base + c * csz
        pltpu.async_copy(col_hbm.at[pl.ds(off, csz)], idx[b], isem[b])
        pltpu.async_copy(h_hbm.at[pl.ds(off, csz), :], rows[b], isem[b])

    def wait(b):
        pltpu.make_async_copy(col_hbm.at[pl.ds(base, csz)], idx[b], isem[b]).wait()
        pltpu.make_async_copy(h_hbm.at[pl.ds(base, csz), :], rows[b], isem[b]).wait()

    def halfstep(c, b):
        @pl.when(c < nch)
        def _():
            wait(b)
            pltpu.sync_copy(rows[b], table_sh.at[idx[b]], add=True)
            @pl.when(c + 2 < nch)
            def _():
                start(c + 2, b)

    start(0, 0)
    @pl.when(nch > 1)
    def _():
        start(1, 1)

    def body(k2, _):
        halfstep(2 * k2, 0)
        halfstep(2 * k2 + 1, 1)
        return _

    lax.fori_loop(0, (nch + 1) // 2, body, 0)


def _gather_x_body(x_hbm, row_hbm, out_hbm, idx0, idx1, rows0, rows1,
                   isem0, isem1, gsem, ssem0, ssem1):
    wid = lax.axis_index("s") * _NC + lax.axis_index("c")
    per_w = _E // _NW
    _gather_pipe(x_hbm, row_hbm, out_hbm, wid * per_w, per_w // _CX, _CX,
                 (idx0, idx1), (rows0, rows1), (isem0, isem1), gsem,
                 (ssem0, ssem1))


_gather_x = functools.partial(
    pl.kernel,
    out_type=jax.ShapeDtypeStruct((_E, _H), jnp.float32),
    mesh=_sc_mesh,
    scratch_types=[
        pltpu.VMEM((_CX,), jnp.int32),
        pltpu.VMEM((_CX,), jnp.int32),
        pltpu.VMEM((_CX, _H), jnp.float32),
        pltpu.VMEM((_CX, _H), jnp.float32),
        pltpu.SemaphoreType.DMA,
        pltpu.SemaphoreType.DMA,
        pltpu.SemaphoreType.DMA,
        pltpu.SemaphoreType.DMA,
        pltpu.SemaphoreType.DMA,
    ],
)(_gather_x_body)


def _sc_layer_body(h_hbm, col_hbm, row_hbm, zeros_hbm, m_hbm,
                   table_sh, idx0, idx1, rows0, rows1,
                   isem0, isem1, gsem, ssem0, ssem1):
    cid = lax.axis_index("c")
    sid = lax.axis_index("s")

    rows_per_tile = _NP // _NS
    pltpu.sync_copy(zeros_hbm.at[pl.ds(sid * rows_per_tile, rows_per_tile), :],
                    table_sh.at[pl.ds(sid * rows_per_tile, rows_per_tile), :])
    plsc.subcore_barrier()

    per_tile = _E // _NS
    _scatter_pipe(h_hbm, col_hbm, table_sh, sid * per_tile, per_tile // _CS,
                  _CS, (idx0, idx1), (rows0, rows1), (isem0, isem1))
    plsc.subcore_barrier()

    wid = sid * _NC + cid
    per_w = _E // _NW
    _gather_pipe(table_sh, row_hbm, m_hbm, wid * per_w, per_w // _CG, _CG,
                 (idx0, idx1), (rows0, rows1), (isem0, isem1), gsem,
                 (ssem0, ssem1))


_sc_layer = functools.partial(
    pl.kernel,
    out_type=jax.ShapeDtypeStruct((_E, _H), jnp.float32),
    mesh=_sc_mesh,
    scratch_types=[
        pltpu.VMEM_SHARED((_NP, _H), jnp.float32),
        pltpu.VMEM((_CS,), jnp.int32),
        pltpu.VMEM((_CS,), jnp.int32),
        pltpu.VMEM((_CS, _H), jnp.float32),
        pltpu.VMEM((_CS, _H), jnp.float32),
        pltpu.SemaphoreType.DMA,
        pltpu.SemaphoreType.DMA,
        pltpu.SemaphoreType.DMA,
        pltpu.SemaphoreType.DMA,
        pltpu.SemaphoreType.DMA,
    ],
)(_sc_layer_body)


def _sc_scatter_body(h_hbm, col_hbm, zeros_hbm, out_hbm, table_sh,
                     idx0, idx1, rows0, rows1, isem0, isem1):
    cid = lax.axis_index("c")
    sid = lax.axis_index("s")

    rows_per_tile = _NP // _NS
    pltpu.sync_copy(zeros_hbm.at[pl.ds(sid * rows_per_tile, rows_per_tile), :],
                    table_sh.at[pl.ds(sid * rows_per_tile, rows_per_tile), :])
    plsc.subcore_barrier()

    wid = sid * _NC + cid
    per_w = _E // _NW
    _scatter_pipe(h_hbm, col_hbm, table_sh, wid * per_w, per_w // _CS, _CS,
                  (idx0, idx1), (rows0, rows1), (isem0, isem1))
    plsc.subcore_barrier()

    pltpu.sync_copy(table_sh.at[pl.ds(sid * rows_per_tile, rows_per_tile), :],
                    out_hbm.at[cid, pl.ds(sid * rows_per_tile, rows_per_tile), :])


_sc_scatter = functools.partial(
    pl.kernel,
    out_type=jax.ShapeDtypeStruct((_NC, _NP, _H), jnp.float32),
    mesh=_sc_mesh,
    scratch_types=[
        pltpu.VMEM_SHARED((_NP, _H), jnp.float32),
        pltpu.VMEM((_CS,), jnp.int32),
        pltpu.VMEM((_CS,), jnp.int32),
        pltpu.VMEM((_CS, _H), jnp.float32),
        pltpu.VMEM((_CS, _H), jnp.float32),
        pltpu.SemaphoreType.DMA,
        pltpu.SemaphoreType.DMA,
    ],
)(_sc_scatter_body)


def _edge_init_body(xg_ref, ea_ref, w_ref, b_ref, out_ref):
    q = jnp.concatenate([xg_ref[...], ea_ref[...]], axis=1)
    acc = jnp.dot(q, w_ref[...], preferred_element_type=jnp.float32)
    out_ref[...] = jnp.maximum(acc + b_ref[...], 0.0)


def _conv_body(m_ref, h_ref, h0_ref, w_ref, b_ref, out_ref):
    h = h_ref[...]
    up = pltpu.roll(h, _BE - 1, 0)
    dn = pltpu.roll(h, 1, 0)
    par = lax.broadcasted_iota(jnp.int32, h.shape, 0) % 2
    rev = jnp.where(par == 0, up, dn)
    t = m_ref[...] - rev
    acc = jnp.dot(t, w_ref[...], preferred_element_type=jnp.float32)
    out_ref[...] = jnp.maximum(acc + b_ref[...] + h0_ref[...], 0.0)


def _final_body(x_ref, p0_ref, p1_ref, bat_ref, w_ref, wf_ref,
                be_ref, bff_ref, out_ref, acc_ref):
    s = p0_ref[...] + p1_ref[...]
    q = jnp.concatenate([x_ref[...], s], axis=1)
    acc = jnp.dot(q, w_ref[...], preferred_element_type=jnp.float32)
    hn = jnp.maximum(acc + be_ref[...], 0.0)
    gi = lax.broadcasted_iota(jnp.int32, (_BN, _G), 1)
    onehot = (bat_ref[...] == gi).astype(jnp.float32)
    pooled = lax.dot_general(onehot, hn, (((0,), (0,)), ((), ())),
                             preferred_element_type=jnp.float32)

    @pl.when(pl.program_id(0) == 0)
    def _():
        acc_ref[...] = jnp.zeros((_G, _H), jnp.float32)

    acc_ref[...] += pooled

    @pl.when(pl.program_id(0) == _N // _BN - 1)
    def _():
        out_ref[...] = (jnp.dot(acc_ref[...], wf_ref[...],
                                preferred_element_type=jnp.float32)
                        + bff_ref[...])


def _edge_block(i):
    return (i, 0)


def _fixed(i):
    return (0, 0)


_eb_spec = pl.BlockSpec((_BE, _H), _edge_block)


def _tc_edge_init(xg, ea, w, b):
    return pl.pallas_call(
        _edge_init_body,
        grid=(_E // _BE,),
        in_specs=[
            _eb_spec,
            pl.BlockSpec((_BE, 16), _edge_block),
            pl.BlockSpec((_H + 16, _H), _fixed),
            pl.BlockSpec((1, _H), _fixed),
        ],
        out_specs=_eb_spec,
        out_shape=jax.ShapeDtypeStruct((_E, _H), jnp.float32),
    )(xg, ea, w, b)


def _tc_conv(m, h, h0, w, b):
    return pl.pallas_call(
        _conv_body,
        grid=(_E // _BE,),
        in_specs=[
            _eb_spec, _eb_spec, _eb_spec,
            pl.BlockSpec((_H, _H), _fixed),
            pl.BlockSpec((1, _H), _fixed),
        ],
        out_specs=_eb_spec,
        out_shape=jax.ShapeDtypeStruct((_E, _H), jnp.float32),
    )(m, h, h0, w, b)


def _tc_final(x, p0, p1, bat2d, w, wf, be, bff):
    nb_spec = pl.BlockSpec((_BN, _H), _edge_block)
    out = pl.pallas_call(
        _final_body,
        grid=(_N // _BN,),
        in_specs=[
            nb_spec, nb_spec, nb_spec,
            pl.BlockSpec((_BN, 1), _edge_block),
            pl.BlockSpec((2 * _H, _H), _fixed),
            pl.BlockSpec((_H, 1), _fixed),
            pl.BlockSpec((1, _H), _fixed),
            pl.BlockSpec((1, 1), _fixed),
        ],
        out_specs=pl.BlockSpec((_G, 1), _fixed),
        out_shape=jax.ShapeDtypeStruct((_G, 1), jnp.float32),
        scratch_shapes=[pltpu.VMEM((_G, _H), jnp.float32)],
    )(x, p0, p1, bat2d, w, wf, be, bff)
    return out.reshape(_G)


def kernel(x, edge_index, edge_attr, batch,
           W_edge_init, b_edge_init,
           W_conv0, b_conv0, W_conv1, b_conv1, W_conv2, b_conv2,
           W_e2n, b_e2n, W_ffn, b_ffn):
    row = edge_index[0]
    col = edge_index[1]
    zeros_n = jnp.zeros((_NP, _H), jnp.float32)

    be0 = b_edge_init.reshape(1, _H)

    xg = _gather_x(x, row)
    h0 = _tc_edge_init(xg, edge_attr, W_edge_init.T, be0)

    h = h0
    for w, b in ((W_conv0, b_conv0), (W_conv1, b_conv1), (W_conv2, b_conv2)):
        m = _sc_layer(h, col, row, zeros_n)
        h = _tc_conv(m, h, h0, w.T, b.reshape(1, _H))

    parts = _sc_scatter(h, col, zeros_n)

    out = _tc_final(x, parts[0, :_N], parts[1, :_N], batch.reshape(_N, 1),
                    W_e2n.T, W_ffn.reshape(_H, 1),
                    b_e2n.reshape(1, _H), b_ffn.reshape(1, 1))
    return out


# speedup vs baseline: 4.0797x; 1.0978x over previous
import functools

import jax
import jax.numpy as jnp
from jax import lax
from jax.experimental import pallas as pl
from jax.experimental.pallas import tpu as pltpu
from jax.experimental.pallas import tpu_sc as plsc

_N = 10000
_NP = 10240
_E = 640000
_H = 128
_G = 512

_NC = 2
_NS = 16
_NW = _NC * _NS

_CS = 160
_CG = 160
_CX = 400

_BE = 1024
_BN = 1000

_sc_mesh = plsc.VectorSubcoreMesh(core_axis_name="c", subcore_axis_name="s")


def _gather_pipe(table, row_hbm, out_hbm, base, nch, csz,
                 idx, rows, isem, gsem, ssem):
    def istart(c, b):
        pltpu.async_copy(row_hbm.at[pl.ds(base + c * csz, csz)], idx[b], isem[b])

    def iwait(b):
        pltpu.make_async_copy(row_hbm.at[pl.ds(base, csz)], idx[b], isem[b]).wait()

    def swait(b):
        pltpu.make_async_copy(rows[b], out_hbm.at[pl.ds(base, csz), :], ssem[b]).wait()

    def halfstep(c, b, k2):
        @pl.when(c < nch)
        def _():
            iwait(b)
            @pl.when(k2 >= 1)
            def _():
                swait(b)
            pltpu.async_copy(table.at[idx[b]], rows[b], gsem).wait()
            @pl.when(c + 2 < nch)
            def _():
                istart(c + 2, b)
            pltpu.async_copy(rows[b], out_hbm.at[pl.ds(base + c * csz, csz), :],
                             ssem[b])

    istart(0, 0)
    @pl.when(nch > 1)
    def _():
        istart(1, 1)

    def body(k2, _):
        halfstep(2 * k2, 0, k2)
        halfstep(2 * k2 + 1, 1, k2)
        return _

    lax.fori_loop(0, (nch + 1) // 2, body, 0)
    swait(0)
    @pl.when(nch > 1)
    def _():
        swait(1)


def _scatter_pipe(h_hbm, col_hbm, table_sh, base, nch, csz,
                  idx, rows, isem):
    def start(c, b):
        off = base + c * csz
        pltpu.async_copy(col_hbm.at[pl.ds(off, csz)], idx[b], isem[b])
        pltpu.async_copy(h_hbm.at[pl.ds(off, csz), :], rows[b], isem[b])

    def wait(b):
        pltpu.make_async_copy(col_hbm.at[pl.ds(base, csz)], idx[b], isem[b]).wait()
        pltpu.make_async_copy(h_hbm.at[pl.ds(base, csz), :], rows[b], isem[b]).wait()

    def halfstep(c, b):
        @pl.when(c < nch)
        def _():
            wait(b)
            pltpu.sync_copy(rows[b], table_sh.at[idx[b]], add=True)
            @pl.when(c + 2 < nch)
            def _():
                start(c + 2, b)

    start(0, 0)
    @pl.when(nch > 1)
    def _():
        start(1, 1)

    def body(k2, _):
        halfstep(2 * k2, 0)
        halfstep(2 * k2 + 1, 1)
        return _

    lax.fori_loop(0, (nch + 1) // 2, body, 0)


def _gather_x_body(x_hbm, row_hbm, out_hbm, idx0, idx1, rows0, rows1,
                   isem0, isem1, gsem, ssem0, ssem1):
    wid = lax.axis_index("s") * _NC + lax.axis_index("c")
    per_w = _E // _NW
    _gather_pipe(x_hbm, row_hbm, out_hbm, wid * per_w, per_w // _CX, _CX,
                 (idx0, idx1), (rows0, rows1), (isem0, isem1), gsem,
                 (ssem0, ssem1))


_gather_x = functools.partial(
    pl.kernel,
    out_type=jax.ShapeDtypeStruct((_E, _H), jnp.float32),
    mesh=_sc_mesh,
    scratch_types=[
        pltpu.VMEM((_CX,), jnp.int32),
        pltpu.VMEM((_CX,), jnp.int32),
        pltpu.VMEM((_CX, _H), jnp.float32),
        pltpu.VMEM((_CX, _H), jnp.float32),
        pltpu.SemaphoreType.DMA,
        pltpu.SemaphoreType.DMA,
        pltpu.SemaphoreType.DMA,
        pltpu.SemaphoreType.DMA,
        pltpu.SemaphoreType.DMA,
    ],
)(_gather_x_body)


def _sc_layer_body(h_hbm, col_hbm, row_hbm, zeros_hbm, m_hbm,
                   table_sh, idx0, idx1, rows0, rows1,
                   isem0, isem1, gsem, ssem0, ssem1):
    cid = lax.axis_index("c")
    sid = lax.axis_index("s")

    rows_per_tile = _NP // _NS
    pltpu.sync_copy(zeros_hbm.at[pl.ds(sid * rows_per_tile, rows_per_tile), :],
                    table_sh.at[pl.ds(sid * rows_per_tile, rows_per_tile), :])
    plsc.subcore_barrier()

    per_tile = _E // _NS
    _scatter_pipe(h_hbm, col_hbm, table_sh, sid * per_tile, per_tile // _CS,
                  _CS, (idx0, idx1), (rows0, rows1), (isem0, isem1))
    plsc.subcore_barrier()

    wid = sid * _NC + cid
    per_w = _E // _NW
    _gather_pipe(table_sh, row_hbm, m_hbm, wid * per_w, per_w // _CG, _CG,
                 (idx0, idx1), (rows0, rows1), (isem0, isem1), gsem,
                 (ssem0, ssem1))


_sc_layer = functools.partial(
    pl.kernel,
    out_type=jax.ShapeDtypeStruct((_E, _H), jnp.float32),
    mesh=_sc_mesh,
    scratch_types=[
        pltpu.VMEM_SHARED((_NP, _H), jnp.float32),
        pltpu.VMEM((_CS,), jnp.int32),
        pltpu.VMEM((_CS,), jnp.int32),
        pltpu.VMEM((_CS, _H), jnp.float32),
        pltpu.VMEM((_CS, _H), jnp.float32),
        pltpu.SemaphoreType.DMA,
        pltpu.SemaphoreType.DMA,
        pltpu.SemaphoreType.DMA,
        pltpu.SemaphoreType.DMA,
        pltpu.SemaphoreType.DMA,
    ],
)(_sc_layer_body)


def _sc_scatter_body(h_hbm, col_hbm, zeros_hbm, out_hbm, table_sh,
                     idx0, idx1, rows0, rows1, isem0, isem1):
    cid = lax.axis_index("c")
    sid = lax.axis_index("s")

    rows_per_tile = _NP // _NS
    pltpu.sync_copy(zeros_hbm.at[pl.ds(sid * rows_per_tile, rows_per_tile), :],
                    table_sh.at[pl.ds(sid * rows_per_tile, rows_per_tile), :])
    plsc.subcore_barrier()

    wid = sid * _NC + cid
    per_w = _E // _NW
    _scatter_pipe(h_hbm, col_hbm, table_sh, wid * per_w, per_w // _CS, _CS,
                  (idx0, idx1), (rows0, rows1), (isem0, isem1))
    plsc.subcore_barrier()

    pltpu.sync_copy(table_sh.at[pl.ds(sid * rows_per_tile, rows_per_tile), :],
                    out_hbm.at[cid, pl.ds(sid * rows_per_tile, rows_per_tile), :])


def _sc_gather2_body(p_hbm, row_hbm, m_hbm, table_sh,
                     idx0, idx1, rows0, rows1,
                     isem0, isem1, gsem, ssem0, ssem1):
    cid = lax.axis_index("c")
    sid = lax.axis_index("s")
    rpt = _NP // _NS

    lane = lax.iota(jnp.int32, 16)

    def cstep(j, _):
        off = sid * rpt + j * _CS
        pltpu.sync_copy(p_hbm.at[0, pl.ds(off, _CS), :],
                        table_sh.at[pl.ds(off, _CS), :])
        pltpu.sync_copy(p_hbm.at[1, pl.ds(off, _CS), :], rows1)

        def fill(t, _2):
            idx0[pl.ds(t * 16, 16)] = off + t * 16 + lane
            return _2

        lax.fori_loop(0, _CS // 16, fill, 0)
        pltpu.sync_copy(rows1, table_sh.at[idx0], add=True)
        return _

    lax.fori_loop(0, rpt // _CS, cstep, 0)
    plsc.subcore_barrier()

    wid = sid * _NC + cid
    per_w = _E // _NW
    _gather_pipe(table_sh, row_hbm, m_hbm, wid * per_w, per_w // _CG, _CG,
                 (idx0, idx1), (rows0, rows1), (isem0, isem1), gsem,
                 (ssem0, ssem1))


_sc_gather2 = functools.partial(
    pl.kernel,
    out_type=jax.ShapeDtypeStruct((_E, _H), jnp.float32),
    mesh=_sc_mesh,
    scratch_types=[
        pltpu.VMEM_SHARED((_NP, _H), jnp.float32),
        pltpu.VMEM((_CS,), jnp.int32),
        pltpu.VMEM((_CS,), jnp.int32),
        pltpu.VMEM((_CS, _H), jnp.float32),
        pltpu.VMEM((_CS, _H), jnp.float32),
        pltpu.SemaphoreType.DMA,
        pltpu.SemaphoreType.DMA,
        pltpu.SemaphoreType.DMA,
        pltpu.SemaphoreType.DMA,
        pltpu.SemaphoreType.DMA,
    ],
)(_sc_gather2_body)


_sc_scatter = functools.partial(
    pl.kernel,
    out_type=jax.ShapeDtypeStruct((_NC, _NP, _H), jnp.float32),
    mesh=_sc_mesh,
    scratch_types=[
        pltpu.VMEM_SHARED((_NP, _H), jnp.float32),
        pltpu.VMEM((_CS,), jnp.int32),
        pltpu.VMEM((_CS,), jnp.int32),
        pltpu.VMEM((_CS, _H), jnp.float32),
        pltpu.VMEM((_CS, _H), jnp.float32),
        pltpu.SemaphoreType.DMA,
        pltpu.SemaphoreType.DMA,
    ],
)(_sc_scatter_body)


def _edge_init_body(xg_ref, ea_ref, w_ref, b_ref, out_ref):
    q = jnp.concatenate([xg_ref[...], ea_ref[...]], axis=1)
    acc = jnp.dot(q, w_ref[...], preferred_element_type=jnp.float32)
    out_ref[...] = jnp.maximum(acc + b_ref[...], 0.0)


def _conv_body(m_ref, h_ref, h0_ref, w_ref, b_ref, out_ref):
    h = h_ref[...]
    up = pltpu.roll(h, _BE - 1, 0)
    dn = pltpu.roll(h, 1, 0)
    par = lax.broadcasted_iota(jnp.int32, h.shape, 0) % 2
    rev = jnp.where(par == 0, up, dn)
    t = m_ref[...] - rev
    acc = jnp.dot(t, w_ref[...], preferred_element_type=jnp.float32)
    out_ref[...] = jnp.maximum(acc + b_ref[...] + h0_ref[...], 0.0)


def _final_body(x_ref, p0_ref, p1_ref, bat_ref, w_ref, wf_ref,
                be_ref, bff_ref, out_ref, acc_ref):
    s = p0_ref[...] + p1_ref[...]
    q = jnp.concatenate([x_ref[...], s], axis=1)
    acc = jnp.dot(q, w_ref[...], preferred_element_type=jnp.float32)
    hn = jnp.maximum(acc + be_ref[...], 0.0)
    gi = lax.broadcasted_iota(jnp.int32, (_BN, _G), 1)
    onehot = (bat_ref[...] == gi).astype(jnp.float32)
    pooled = lax.dot_general(onehot, hn, (((0,), (0,)), ((), ())),
                             preferred_element_type=jnp.float32)

    @pl.when(pl.program_id(0) == 0)
    def _():
        acc_ref[...] = jnp.zeros((_G, _H), jnp.float32)

    acc_ref[...] += pooled

    @pl.when(pl.program_id(0) == _N // _BN - 1)
    def _():
        out_ref[...] = (jnp.dot(acc_ref[...], wf_ref[...],
                                preferred_element_type=jnp.float32)
                        + bff_ref[...])


def _edge_block(i):
    return (i, 0)


def _fixed(i):
    return (0, 0)


_eb_spec = pl.BlockSpec((_BE, _H), _edge_block)


def _tc_edge_init(xg, ea, w, b):
    return pl.pallas_call(
        _edge_init_body,
        grid=(_E // _BE,),
        in_specs=[
            _eb_spec,
            pl.BlockSpec((_BE, 16), _edge_block),
            pl.BlockSpec((_H + 16, _H), _fixed),
            pl.BlockSpec((1, _H), _fixed),
        ],
        out_specs=_eb_spec,
        out_shape=jax.ShapeDtypeStruct((_E, _H), jnp.float32),
    )(xg, ea, w, b)


def _tc_conv(m, h, h0, w, b):
    return pl.pallas_call(
        _conv_body,
        grid=(_E // _BE,),
        in_specs=[
            _eb_spec, _eb_spec, _eb_spec,
            pl.BlockSpec((_H, _H), _fixed),
            pl.BlockSpec((1, _H), _fixed),
        ],
        out_specs=_eb_spec,
        out_shape=jax.ShapeDtypeStruct((_E, _H), jnp.float32),
    )(m, h, h0, w, b)


def _tc_final(x, p0, p1, bat2d, w, wf, be, bff):
    nb_spec = pl.BlockSpec((_BN, _H), _edge_block)
    out = pl.pallas_call(
        _final_body,
        grid=(_N // _BN,),
        in_specs=[
            nb_spec, nb_spec, nb_spec,
            pl.BlockSpec((_BN, 1), _edge_block),
            pl.BlockSpec((2 * _H, _H), _fixed),
            pl.BlockSpec((_H, 1), _fixed),
            pl.BlockSpec((1, _H), _fixed),
            pl.BlockSpec((1, 1), _fixed),
        ],
        out_specs=pl.BlockSpec((_G, 1), _fixed),
        out_shape=jax.ShapeDtypeStruct((_G, 1), jnp.float32),
        scratch_shapes=[pltpu.VMEM((_G, _H), jnp.float32)],
    )(x, p0, p1, bat2d, w, wf, be, bff)
    return out.reshape(_G)


def kernel(x, edge_index, edge_attr, batch,
           W_edge_init, b_edge_init,
           W_conv0, b_conv0, W_conv1, b_conv1, W_conv2, b_conv2,
           W_e2n, b_e2n, W_ffn, b_ffn):
    row = edge_index[0]
    col = edge_index[1]
    zeros_n = jnp.zeros((_NP, _H), jnp.float32)

    be0 = b_edge_init.reshape(1, _H)

    xg = _gather_x(x, row)
    h0 = _tc_edge_init(xg, edge_attr, W_edge_init.T, be0)

    h = h0
    for w, b in ((W_conv0, b_conv0), (W_conv1, b_conv1), (W_conv2, b_conv2)):
        parts_l = _sc_scatter(h, col, zeros_n)
        m = _sc_gather2(parts_l, row)
        h = _tc_conv(m, h, h0, w.T, b.reshape(1, _H))

    parts = _sc_scatter(h, col, zeros_n)

    out = _tc_final(x, parts[0, :_N], parts[1, :_N], batch.reshape(_N, 1),
                    W_e2n.T, W_ffn.reshape(_H, 1),
                    b_e2n.reshape(1, _H), b_ffn.reshape(1, 1))
    return out


# speedup vs baseline: 4.2062x; 1.0310x over previous
import functools

import jax
import jax.numpy as jnp
from jax import lax
from jax.experimental import pallas as pl
from jax.experimental.pallas import tpu as pltpu
from jax.experimental.pallas import tpu_sc as plsc

_N = 10000
_NP = 10240
_E = 640000
_H = 128
_G = 512

_NC = 2
_NS = 16
_NW = _NC * _NS

_CS = 160
_CG = 160
_CX = 400

_BE = 1024
_BN = 1000

_sc_mesh = plsc.VectorSubcoreMesh(core_axis_name="c", subcore_axis_name="s")


def _gather_pipe(table, row_hbm, out_hbm, base, nch, csz,
                 idx, rows, isem, gsem, ssem):
    def istart(c, b):
        pltpu.async_copy(row_hbm.at[pl.ds(base + c * csz, csz)], idx[b], isem[b])

    def iwait(b):
        pltpu.make_async_copy(row_hbm.at[pl.ds(base, csz)], idx[b], isem[b]).wait()

    def swait(b):
        pltpu.make_async_copy(rows[b], out_hbm.at[pl.ds(base, csz), :], ssem[b]).wait()

    def halfstep(c, b, k2):
        @pl.when(c < nch)
        def _():
            iwait(b)
            @pl.when(k2 >= 1)
            def _():
                swait(b)
            pltpu.async_copy(table.at[idx[b]], rows[b], gsem).wait()
            @pl.when(c + 2 < nch)
            def _():
                istart(c + 2, b)
            pltpu.async_copy(rows[b], out_hbm.at[pl.ds(base + c * csz, csz), :],
                             ssem[b])

    istart(0, 0)
    @pl.when(nch > 1)
    def _():
        istart(1, 1)

    def body(k2, _):
        halfstep(2 * k2, 0, k2)
        halfstep(2 * k2 + 1, 1, k2)
        return _

    lax.fori_loop(0, (nch + 1) // 2, body, 0)
    swait(0)
    @pl.when(nch > 1)
    def _():
        swait(1)


def _scatter_pipe(h_hbm, col_hbm, table_sh, base, nch, csz,
                  idx, rows, isem):
    def start(c, b):
        off = base + c * csz
        pltpu.async_copy(col_hbm.at[pl.ds(off, csz)], idx[b], isem[b])
        pltpu.async_copy(h_hbm.at[pl.ds(off, csz), :], rows[b], isem[b])

    def wait(b):
        pltpu.make_async_copy(col_hbm.at[pl.ds(base, csz)], idx[b], isem[b]).wait()
        pltpu.make_async_copy(h_hbm.at[pl.ds(base, csz), :], rows[b], isem[b]).wait()

    def halfstep(c, b):
        @pl.when(c < nch)
        def _():
            wait(b)
            pltpu.sync_copy(rows[b], table_sh.at[idx[b]], add=True)
            @pl.when(c + 2 < nch)
            def _():
                start(c + 2, b)

    start(0, 0)
    @pl.when(nch > 1)
    def _():
        start(1, 1)

    def body(k2, _):
        halfstep(2 * k2, 0)
        halfstep(2 * k2 + 1, 1)
        return _

    lax.fori_loop(0, (nch + 1) // 2, body, 0)


def _gather_x_body(x_hbm, row_hbm, out_hbm, idx0, idx1, rows0, rows1,
                   isem0, isem1, gsem, ssem0, ssem1):
    wid = lax.axis_index("s") * _NC + lax.axis_index("c")
    per_w = _E // _NW
    _gather_pipe(x_hbm, row_hbm, out_hbm, wid * per_w, per_w // _CX, _CX,
                 (idx0, idx1), (rows0, rows1), (isem0, isem1), gsem,
                 (ssem0, ssem1))


_gather_x = functools.partial(
    pl.kernel,
    out_type=jax.ShapeDtypeStruct((_E, _H), jnp.float32),
    mesh=_sc_mesh,
    scratch_types=[
        pltpu.VMEM((_CX,), jnp.int32),
        pltpu.VMEM((_CX,), jnp.int32),
        pltpu.VMEM((_CX, _H), jnp.float32),
        pltpu.VMEM((_CX, _H), jnp.float32),
        pltpu.SemaphoreType.DMA,
        pltpu.SemaphoreType.DMA,
        pltpu.SemaphoreType.DMA,
        pltpu.SemaphoreType.DMA,
        pltpu.SemaphoreType.DMA,
    ],
)(_gather_x_body)


def _sc_layer_body(h_hbm, col_hbm, row_hbm, zeros_hbm, m_hbm,
                   table_sh, idx0, idx1, rows0, rows1,
                   isem0, isem1, gsem, ssem0, ssem1):
    cid = lax.axis_index("c")
    sid = lax.axis_index("s")

    rows_per_tile = _NP // _NS
    pltpu.sync_copy(zeros_hbm.at[pl.ds(sid * rows_per_tile, rows_per_tile), :],
                    table_sh.at[pl.ds(sid * rows_per_tile, rows_per_tile), :])
    plsc.subcore_barrier()

    per_tile = _E // _NS
    _scatter_pipe(h_hbm, col_hbm, table_sh, sid * per_tile, per_tile // _CS,
                  _CS, (idx0, idx1), (rows0, rows1), (isem0, isem1))
    plsc.subcore_barrier()

    wid = sid * _NC + cid
    per_w = _E // _NW
    _gather_pipe(table_sh, row_hbm, m_hbm, wid * per_w, per_w // _CG, _CG,
                 (idx0, idx1), (rows0, rows1), (isem0, isem1), gsem,
                 (ssem0, ssem1))


_sc_layer = functools.partial(
    pl.kernel,
    out_type=jax.ShapeDtypeStruct((_E, _H), jnp.float32),
    mesh=_sc_mesh,
    scratch_types=[
        pltpu.VMEM_SHARED((_NP, _H), jnp.float32),
        pltpu.VMEM((_CS,), jnp.int32),
        pltpu.VMEM((_CS,), jnp.int32),
        pltpu.VMEM((_CS, _H), jnp.float32),
        pltpu.VMEM((_CS, _H), jnp.float32),
        pltpu.SemaphoreType.DMA,
        pltpu.SemaphoreType.DMA,
        pltpu.SemaphoreType.DMA,
        pltpu.SemaphoreType.DMA,
        pltpu.SemaphoreType.DMA,
    ],
)(_sc_layer_body)


def _sc_scatter_body(h_hbm, col_hbm, zeros_hbm, out_hbm, table_sh,
                     idx0, idx1, rows0, rows1, isem0, isem1):
    cid = lax.axis_index("c")
    sid = lax.axis_index("s")

    rows_per_tile = _NP // _NS
    pltpu.sync_copy(zeros_hbm.at[pl.ds(sid * rows_per_tile, rows_per_tile), :],
                    table_sh.at[pl.ds(sid * rows_per_tile, rows_per_tile), :])
    plsc.subcore_barrier()

    wid = sid * _NC + cid
    per_w = _E // _NW
    _scatter_pipe(h_hbm, col_hbm, table_sh, wid * per_w, per_w // _CS, _CS,
                  (idx0, idx1), (rows0, rows1), (isem0, isem1))
    plsc.subcore_barrier()

    pltpu.sync_copy(table_sh.at[pl.ds(sid * rows_per_tile, rows_per_tile), :],
                    out_hbm.at[cid, pl.ds(sid * rows_per_tile, rows_per_tile), :])


def _sc_gather2_body(p_hbm, row_hbm, m_hbm, table_sh,
                     idx0, idx1, rows0, rows1,
                     isem0, isem1, gsem, ssem0, ssem1):
    cid = lax.axis_index("c")
    sid = lax.axis_index("s")
    rpt = _NP // _NS

    lane = lax.iota(jnp.int32, 16)

    def cstep(j, _):
        off = sid * rpt + j * _CS
        pltpu.sync_copy(p_hbm.at[0, pl.ds(off, _CS), :],
                        table_sh.at[pl.ds(off, _CS), :])
        pltpu.sync_copy(p_hbm.at[1, pl.ds(off, _CS), :], rows1)

        def fill(t, _2):
            idx0[pl.ds(t * 16, 16)] = off + t * 16 + lane
            return _2

        lax.fori_loop(0, _CS // 16, fill, 0)
        pltpu.sync_copy(rows1, table_sh.at[idx0], add=True)
        return _

    lax.fori_loop(0, rpt // _CS, cstep, 0)
    plsc.subcore_barrier()

    wid = sid * _NC + cid
    per_w = _E // _NW
    _gather_pipe(table_sh, row_hbm, m_hbm, wid * per_w, per_w // _CG, _CG,
                 (idx0, idx1), (rows0, rows1), (isem0, isem1), gsem,
                 (ssem0, ssem1))


_sc_gather2 = functools.partial(
    pl.kernel,
    out_type=jax.ShapeDtypeStruct((_E, _H), jnp.float32),
    mesh=_sc_mesh,
    scratch_types=[
        pltpu.VMEM_SHARED((_NP, _H), jnp.float32),
        pltpu.VMEM((_CS,), jnp.int32),
        pltpu.VMEM((_CS,), jnp.int32),
        pltpu.VMEM((_CS, _H), jnp.float32),
        pltpu.VMEM((_CS, _H), jnp.float32),
        pltpu.SemaphoreType.DMA,
        pltpu.SemaphoreType.DMA,
        pltpu.SemaphoreType.DMA,
        pltpu.SemaphoreType.DMA,
        pltpu.SemaphoreType.DMA,
    ],
)(_sc_gather2_body)


_sc_scatter = functools.partial(
    pl.kernel,
    out_type=jax.ShapeDtypeStruct((_NC, _NP, _H), jnp.float32),
    mesh=_sc_mesh,
    scratch_types=[
        pltpu.VMEM_SHARED((_NP, _H), jnp.float32),
        pltpu.VMEM((_CS,), jnp.int32),
        pltpu.VMEM((_CS,), jnp.int32),
        pltpu.VMEM((_CS, _H), jnp.float32),
        pltpu.VMEM((_CS, _H), jnp.float32),
        pltpu.SemaphoreType.DMA,
        pltpu.SemaphoreType.DMA,
    ],
)(_sc_scatter_body)


def _edge_init_body(xg_ref, ea_ref, w_ref, b_ref, out_ref):
    q = jnp.concatenate([xg_ref[...], ea_ref[...]], axis=1)
    acc = jnp.dot(q, w_ref[...], preferred_element_type=jnp.float32)
    out_ref[...] = jnp.maximum(acc + b_ref[...], 0.0)


def _conv_body(m_ref, h_ref, h0_ref, w_ref, b_ref, out_ref):
    h = h_ref[...]
    up = pltpu.roll(h, _BE - 1, 0)
    dn = pltpu.roll(h, 1, 0)
    par = lax.broadcasted_iota(jnp.int32, h.shape, 0) % 2
    rev = jnp.where(par == 0, up, dn)
    t = m_ref[...] - rev
    acc = jnp.dot(t, w_ref[...], preferred_element_type=jnp.float32)
    out_ref[...] = jnp.maximum(acc + b_ref[...] + h0_ref[...], 0.0)


def _final_body(x_ref, p0_ref, p1_ref, bat_ref, w_ref, wf_ref,
                be_ref, bff_ref, out_ref, acc_ref):
    s = p0_ref[...] + p1_ref[...]
    q = jnp.concatenate([x_ref[...], s], axis=1)
    acc = jnp.dot(q, w_ref[...], preferred_element_type=jnp.float32)
    hn = jnp.maximum(acc + be_ref[...], 0.0)
    gi = lax.broadcasted_iota(jnp.int32, (_BN, _G), 1)
    onehot = (bat_ref[...] == gi).astype(jnp.float32)
    pooled = lax.dot_general(onehot, hn, (((0,), (0,)), ((), ())),
                             preferred_element_type=jnp.float32)

    @pl.when(pl.program_id(0) == 0)
    def _():
        acc_ref[...] = jnp.zeros((_G, _H), jnp.float32)

    acc_ref[...] += pooled

    @pl.when(pl.program_id(0) == _N // _BN - 1)
    def _():
        out_ref[...] = (jnp.dot(acc_ref[...], wf_ref[...],
                                preferred_element_type=jnp.float32)
                        + bff_ref[...])


def _edge_block(i):
    return (i, 0)


def _fixed(i):
    return (0, 0)


_eb_spec = pl.BlockSpec((_BE, _H), _edge_block)


def _tc_edge_init(xg, ea, w, b):
    return pl.pallas_call(
        _edge_init_body,
        grid=(_E // _BE,),
        in_specs=[
            _eb_spec,
            pl.BlockSpec((_BE, 16), _edge_block),
            pl.BlockSpec((_H + 16, _H), _fixed),
            pl.BlockSpec((1, _H), _fixed),
        ],
        out_specs=_eb_spec,
        out_shape=jax.ShapeDtypeStruct((_E, _H), jnp.float32),
    )(xg, ea, w, b)


def _tc_conv(m, h, h0, w, b):
    return pl.pallas_call(
        _conv_body,
        grid=(_E // _BE,),
        in_specs=[
            _eb_spec, _eb_spec, _eb_spec,
            pl.BlockSpec((_H, _H), _fixed),
            pl.BlockSpec((1, _H), _fixed),
        ],
        out_specs=_eb_spec,
        out_shape=jax.ShapeDtypeStruct((_E, _H), jnp.float32),
    )(m, h, h0, w, b)


def _tc_final(x, p0, p1, bat2d, w, wf, be, bff):
    nb_spec = pl.BlockSpec((_BN, _H), _edge_block)
    out = pl.pallas_call(
        _final_body,
        grid=(_N // _BN,),
        in_specs=[
            nb_spec, nb_spec, nb_spec,
            pl.BlockSpec((_BN, 1), _edge_block),
            pl.BlockSpec((2 * _H, _H), _fixed),
            pl.BlockSpec((_H, 1), _fixed),
            pl.BlockSpec((1, _H), _fixed),
            pl.BlockSpec((1, 1), _fixed),
        ],
        out_specs=pl.BlockSpec((_G, 1), _fixed),
        out_shape=jax.ShapeDtypeStruct((_G, 1), jnp.float32),
        scratch_shapes=[pltpu.VMEM((_G, _H), jnp.float32)],
    )(x, p0, p1, bat2d, w, wf, be, bff)
    return out.reshape(_G)


def kernel(x, edge_index, edge_attr, batch,
           W_edge_init, b_edge_init,
           W_conv0, b_conv0, W_conv1, b_conv1, W_conv2, b_conv2,
           W_e2n, b_e2n, W_ffn, b_ffn):
    row = edge_index[0]
    col = edge_index[1]
    zeros_n = jnp.zeros((_NP, _H), jnp.float32)

    be0 = b_edge_init.reshape(1, _H)

    x_pad = jnp.concatenate(
        [x, jnp.zeros((_NP - _N, _H), jnp.float32)], axis=0)
    xg = _sc_gather2(jnp.stack([x_pad, jnp.zeros((_NP, _H), jnp.float32)]), row)
    h0 = _tc_edge_init(xg, edge_attr, W_edge_init.T, be0)

    h = h0
    for w, b in ((W_conv0, b_conv0), (W_conv1, b_conv1), (W_conv2, b_conv2)):
        parts_l = _sc_scatter(h, col, zeros_n)
        m = _sc_gather2(parts_l, row)
        h = _tc_conv(m, h, h0, w.T, b.reshape(1, _H))

    parts = _sc_scatter(h, col, zeros_n)

    out = _tc_final(x, parts[0, :_N], parts[1, :_N], batch.reshape(_N, 1),
                    W_e2n.T, W_ffn.reshape(_H, 1),
                    b_e2n.reshape(1, _H), b_ffn.reshape(1, 1))
    return out
